# baseline (speedup 1.0000x reference)
import jax
import jax.numpy as jnp
from jax.experimental import pallas as pl
from jax.experimental.pallas import tpu as pltpu

NG = 128
ROWS = 16384


def _body(x_ref, ids_ref, W1_ref, b1_ref, W2_ref, b2_ref, n_ref, out_ref,
          d_ref, acc_ref):
    k = pl.program_id(0)
    nb = pl.num_programs(0)

    @pl.when(k == 0)
    def _init():
        d_ref[...] = jnp.zeros(d_ref.shape, jnp.float32)
        acc_ref[...] = jnp.zeros(acc_ref.shape, jnp.float32)

    x = x_ref[...]
    valid = jnp.minimum(n_ref[0, 0] - k * ROWS, ROWS).astype(jnp.int16)
    row16 = jax.lax.broadcasted_iota(jnp.int16, (ROWS, 128), 0)
    xb = jnp.where(row16 < valid, x.astype(jnp.bfloat16), jnp.bfloat16(0.0))
    ids = ids_ref[...].astype(jnp.int16).reshape(1, ROWS)
    w2 = W2_ref[...]
    h = jnp.tanh(jnp.dot(xb, W1_ref[...], preferred_element_type=jnp.float32)
                 + b1_ref[...])
    b2e = b2_ref[0, 0] - jnp.sum(jnp.abs(w2.astype(jnp.float32)))
    sT = (jax.lax.dot_general(w2, h.astype(jnp.bfloat16),
                              (((0,), (1,)), ((), ())),
                              preferred_element_type=jnp.float32)
          + b2e)
    col = jax.lax.broadcasted_iota(jnp.int32, (1, ROWS), 1)
    ebT = jnp.where(col < (n_ref[0, 0] - k * ROWS), jnp.exp(sT),
                    0.0).astype(jnp.bfloat16)

    g_iota = jax.lax.broadcasted_iota(jnp.int16, (NG, ROWS), 0)
    emT = jnp.where(ids == g_iota, ebT, jnp.bfloat16(0.0))
    ones_c = jnp.ones((ROWS, 1), jnp.bfloat16)
    d_ref[...] += jnp.dot(emT, ones_c, preferred_element_type=jnp.float32)
    acc_ref[...] += jnp.dot(emT, xb, preferred_element_type=jnp.float32)

    @pl.when(k == nb - 1)
    def _fin():
        dcol = d_ref[...]
        out_ref[...] = jnp.where(dcol == 0.0, 0.0, acc_ref[...] / dcol)


def kernel(x, batch, W1, b1, W2, b2):
    N, d = x.shape
    nb = (N + ROWS - 1) // ROWS
    b1r = b1.reshape(1, -1)
    b2r = b2.reshape(1, 1)
    return pl.pallas_call(
        _body,
        grid=(nb,),
        in_specs=[
            pl.BlockSpec((ROWS, d), lambda k: (k, 0)),
            pl.BlockSpec((ROWS,), lambda k: (k,)),
            pl.BlockSpec((d, d // 2), lambda k: (0, 0)),
            pl.BlockSpec((1, d // 2), lambda k: (0, 0)),
            pl.BlockSpec((d // 2, 1), lambda k: (0, 0)),
            pl.BlockSpec((1, 1), lambda k: (0, 0)),
            pl.BlockSpec((1, 1), lambda k: (0, 0)),
        ],
        out_specs=pl.BlockSpec((NG, d), lambda k: (0, 0)),
        out_shape=jax.ShapeDtypeStruct((NG, d), jnp.float32),
        scratch_shapes=[
            pltpu.VMEM((NG, 1), jnp.float32),
            pltpu.VMEM((NG, d), jnp.float32),
        ],
        compiler_params=pltpu.CompilerParams(
            dimension_semantics=("arbitrary",)),
    )(x, batch, W1.astype(jnp.bfloat16), b1r, W2.astype(jnp.bfloat16), b2r,
      jnp.full((1, 1), N, jnp.int32))


# speedup vs baseline: 1.0347x; 1.0347x over previous
import jax
import jax.numpy as jnp
from jax.experimental import pallas as pl
from jax.experimental.pallas import tpu as pltpu

NG = 128
ROWS = 8192


def _body(x_ref, ids_ref, W1_ref, b1_ref, W2_ref, b2_ref, n_ref, out_ref,
          d_ref, acc_ref):
    k = pl.program_id(0)
    nb = pl.num_programs(0)

    @pl.when(k == 0)
    def _init():
        d_ref[...] = jnp.zeros(d_ref.shape, jnp.float32)
        acc_ref[...] = jnp.zeros(acc_ref.shape, jnp.float32)

    x = x_ref[...]
    valid = jnp.minimum(n_ref[0, 0] - k * ROWS, ROWS).astype(jnp.int16)
    row16 = jax.lax.broadcasted_iota(jnp.int16, (ROWS, 128), 0)
    xb = jnp.where(row16 < valid, x.astype(jnp.bfloat16), jnp.bfloat16(0.0))
    ids = ids_ref[...].astype(jnp.int16).reshape(1, ROWS)
    w2 = W2_ref[...]
    h = jnp.tanh(jnp.dot(xb, W1_ref[...], preferred_element_type=jnp.float32)
                 + b1_ref[...])
    b2e = b2_ref[0, 0] - jnp.sum(jnp.abs(w2.astype(jnp.float32)))
    sT = (jax.lax.dot_general(w2, h.astype(jnp.bfloat16),
                              (((0,), (1,)), ((), ())),
                              preferred_element_type=jnp.float32)
          + b2e)
    col = jax.lax.broadcasted_iota(jnp.int32, (1, ROWS), 1)
    ebT = jnp.where(col < (n_ref[0, 0] - k * ROWS), jnp.exp(sT),
                    0.0).astype(jnp.bfloat16)

    g_iota = jax.lax.broadcasted_iota(jnp.int16, (NG, ROWS), 0)
    emT = jnp.where(ids == g_iota, ebT, jnp.bfloat16(0.0))
    ones_c = jnp.ones((ROWS, 1), jnp.bfloat16)
    d_ref[...] += jnp.dot(emT, ones_c, preferred_element_type=jnp.float32)
    acc_ref[...] += jnp.dot(emT, xb, preferred_element_type=jnp.float32)

    @pl.when(k == nb - 1)
    def _fin():
        dcol = d_ref[...]
        out_ref[...] = jnp.where(dcol == 0.0, 0.0, acc_ref[...] / dcol)


def kernel(x, batch, W1, b1, W2, b2):
    N, d = x.shape
    nb = (N + ROWS - 1) // ROWS
    b2r = b2.reshape(1, 1)
    return pl.pallas_call(
        _body,
        grid=(nb,),
        in_specs=[
            pl.BlockSpec((ROWS, d), lambda k: (k, 0)),
            pl.BlockSpec((ROWS,), lambda k: (k,)),
            pl.BlockSpec((d, d // 2), lambda k: (0, 0)),
            pl.BlockSpec((1, d // 2), lambda k: (0, 0)),
            pl.BlockSpec((d // 2, 1), lambda k: (0, 0)),
            pl.BlockSpec((1, 1), lambda k: (0, 0)),
            pl.BlockSpec((1, 1), lambda k: (0, 0)),
        ],
        out_specs=pl.BlockSpec((NG, d), lambda k: (0, 0)),
        out_shape=jax.ShapeDtypeStruct((NG, d), jnp.float32),
        scratch_shapes=[
            pltpu.VMEM((NG, 1), jnp.float32),
            pltpu.VMEM((NG, d), jnp.float32),
        ],
        compiler_params=pltpu.CompilerParams(
            dimension_semantics=("arbitrary",)),
    )(x, batch, W1.astype(jnp.bfloat16), b1.reshape(1, -1),
      W2.astype(jnp.bfloat16), b2r,
      jnp.full((1, 1), N, jnp.int32))


# speedup vs baseline: 1.1691x; 1.1298x over previous
import jax
import jax.numpy as jnp
from jax.experimental import pallas as pl
from jax.experimental.pallas import tpu as pltpu

NG = 128
ROWS = 8192


def _body(x_ref, ids_ref, W1_ref, b1_ref, W2_ref, b2_ref, n_ref, out_ref,
          d_ref, acc_ref):
    k = pl.program_id(0)
    nb = pl.num_programs(0)

    @pl.when(k == 0)
    def _init():
        d_ref[...] = jnp.zeros(d_ref.shape, jnp.float32)
        acc_ref[...] = jnp.zeros(acc_ref.shape, jnp.float32)

    x = x_ref[...]
    valid = jnp.minimum(n_ref[0, 0] - k * ROWS, ROWS).astype(jnp.int16)
    row16 = jax.lax.broadcasted_iota(jnp.int16, (ROWS, 128), 0)
    xb = jnp.where(row16 < valid, x.astype(jnp.bfloat16), jnp.bfloat16(0.0))
    ids = ids_ref[...].astype(jnp.int16).reshape(1, ROWS)
    w2 = W2_ref[...]
    h = jnp.tanh(jnp.dot(xb, W1_ref[...], preferred_element_type=jnp.float32)
                 + b1_ref[...])
    b2e = b2_ref[0, 0] - jnp.sum(jnp.abs(w2.astype(jnp.float32)))
    sT = (jax.lax.dot_general(w2, h.astype(jnp.bfloat16),
                              (((0,), (1,)), ((), ())),
                              preferred_element_type=jnp.float32)
          + b2e)
    col = jax.lax.broadcasted_iota(jnp.int32, (1, ROWS), 1)
    ebT = jnp.where(col < (n_ref[0, 0] - k * ROWS), jnp.exp(sT),
                    0.0).astype(jnp.bfloat16)

    g_iota = jax.lax.broadcasted_iota(jnp.int16, (NG, ROWS), 0)
    emT = jnp.where(ids == g_iota, ebT, jnp.bfloat16(0.0))
    d_ref[...] += jnp.sum(emT.astype(jnp.float32), axis=1, keepdims=True)
    acc_ref[...] += jnp.dot(emT, xb, preferred_element_type=jnp.float32)

    @pl.when(k == nb - 1)
    def _fin():
        dcol = d_ref[...]
        out_ref[...] = jnp.where(dcol == 0.0, 0.0, acc_ref[...] / dcol)


def kernel(x, batch, W1, b1, W2, b2):
    N, d = x.shape
    nb = (N + ROWS - 1) // ROWS
    b2r = b2.reshape(1, 1)
    return pl.pallas_call(
        _body,
        grid=(nb,),
        in_specs=[
            pl.BlockSpec((ROWS, d), lambda k: (k, 0)),
            pl.BlockSpec((ROWS,), lambda k: (k,)),
            pl.BlockSpec((d, d // 2), lambda k: (0, 0)),
            pl.BlockSpec((1, d // 2), lambda k: (0, 0)),
            pl.BlockSpec((d // 2, 1), lambda k: (0, 0)),
            pl.BlockSpec((1, 1), lambda k: (0, 0)),
            pl.BlockSpec((1, 1), lambda k: (0, 0)),
        ],
        out_specs=pl.BlockSpec((NG, d), lambda k: (0, 0)),
        out_shape=jax.ShapeDtypeStruct((NG, d), jnp.float32),
        scratch_shapes=[
            pltpu.VMEM((NG, 1), jnp.float32),
            pltpu.VMEM((NG, d), jnp.float32),
        ],
        compiler_params=pltpu.CompilerParams(
            dimension_semantics=("arbitrary",)),
    )(x, batch, W1.astype(jnp.bfloat16), b1.reshape(1, -1),
      W2.astype(jnp.bfloat16), b2r,
      jnp.full((1, 1), N, jnp.int32))
